# baseline (speedup 1.0000x reference)
import functools

import jax
import jax.numpy as jnp
from jax import lax
from jax.experimental import pallas as pl
from jax.experimental.pallas import tpu as pltpu
from jax.experimental.pallas import tpu_sc as plsc


def _sc_gather(table, idx):
    V, D = table.shape
    B = idx.shape[0]
    info = plsc.get_sparse_core_info()
    NC, NS = info.num_cores, info.num_subcores
    NW = NC * NS
    b_per_w = B // NW
    mesh = plsc.VectorSubcoreMesh(core_axis_name="c", subcore_axis_name="s")

    @functools.partial(
        pl.kernel,
        mesh=mesh,
        out_type=jax.ShapeDtypeStruct((B, D), jnp.float32),
        scratch_types=[
            pltpu.VMEM((b_per_w,), jnp.int32),
            pltpu.VMEM((b_per_w, D), jnp.float32),
            pltpu.SemaphoreType.DMA,
        ],
        compiler_params=pltpu.CompilerParams(use_tc_tiling_on_sc=False),
    )
    def gather_k(table_hbm, idx_hbm, out_hbm, idx_v, rows_v, sem):
        wid = lax.axis_index("s") * NC + lax.axis_index("c")
        base = wid * b_per_w
        pltpu.sync_copy(idx_hbm.at[pl.ds(base, b_per_w)], idx_v)
        pltpu.async_copy(table_hbm.at[idx_v], rows_v, sem).wait()
        pltpu.sync_copy(rows_v, out_hbm.at[pl.ds(base, b_per_w)])

    return gather_k(table, idx)


_BM = 8
_NBUF = 12
_NTHREAD = 2


def _tc_project(latent, W, b):
    B, D = latent.shape
    V = W.shape[0]
    Wt = W.T
    b2 = b.reshape(1, V)
    grid = B // _BM

    def body(lat_ref, wt_ref, b_ref, out_hbm, bufs, sems):
        i = pl.program_id(0)
        slot = lax.rem(i, _NBUF)
        acc = lax.dot_general(
            lat_ref[pl.ds(i * _BM, _BM), :], wt_ref[...],
            (((1,), (0,)), ((), ())),
            preferred_element_type=jnp.float32,
        ) + b_ref[...]

        @pl.when(i >= _NBUF)
        def _():
            pltpu.make_async_copy(
                bufs.at[slot],
                out_hbm.at[pl.ds((i - _NBUF) * _BM, _BM), :],
                sems.at[slot],
            ).wait()

        bufs[slot] = acc
        for k in range(_NTHREAD):
            @pl.when(lax.rem(i, _NTHREAD) == k)
            def _():
                pltpu.make_async_copy(
                    bufs.at[slot],
                    out_hbm.at[pl.ds(i * _BM, _BM), :],
                    sems.at[slot],
                ).start(priority=k)

        @pl.when(i == grid - 1)
        def _():
            for s in range(max(0, grid - _NBUF), grid):
                sl = s % _NBUF
                pltpu.make_async_copy(
                    bufs.at[sl],
                    out_hbm.at[pl.ds(s * _BM, _BM), :],
                    sems.at[sl],
                ).wait()

    return pl.pallas_call(
        body,
        grid=(grid,),
        in_specs=[
            pl.BlockSpec((B, D), lambda i: (0, 0)),
            pl.BlockSpec((D, V), lambda i: (0, 0)),
            pl.BlockSpec((1, V), lambda i: (0, 0)),
        ],
        out_specs=pl.BlockSpec(memory_space=pl.ANY),
        out_shape=jax.ShapeDtypeStruct((B, V), jnp.float32),
        scratch_shapes=[
            pltpu.VMEM((_NBUF, _BM, V), jnp.float32),
            pltpu.SemaphoreType.DMA((_NBUF,)),
        ],
        compiler_params=pltpu.CompilerParams(
            vmem_limit_bytes=110 * 1024 * 1024,
        ),
    )(latent, Wt, b2)


def kernel(inputs, emb_table, W, b):
    idx = inputs.astype(jnp.int32)
    latent = _sc_gather(emb_table, idx)
    return _tc_project(latent, W, b)


# speedup vs baseline: 1.0114x; 1.0114x over previous
import functools

import jax
import jax.numpy as jnp
from jax import lax
from jax.experimental import pallas as pl
from jax.experimental.pallas import tpu as pltpu
from jax.experimental.pallas import tpu_sc as plsc


def _sc_gather(table, idx):
    V, D = table.shape
    B = idx.shape[0]
    info = plsc.get_sparse_core_info()
    NC, NS = info.num_cores, info.num_subcores
    NW = NC * NS
    b_per_w = B // NW
    mesh = plsc.VectorSubcoreMesh(core_axis_name="c", subcore_axis_name="s")

    @functools.partial(
        pl.kernel,
        mesh=mesh,
        out_type=jax.ShapeDtypeStruct((B, D), jnp.float32),
        scratch_types=[
            pltpu.VMEM((b_per_w,), jnp.int32),
            pltpu.VMEM((b_per_w, D), jnp.float32),
            pltpu.SemaphoreType.DMA,
        ],
        compiler_params=pltpu.CompilerParams(use_tc_tiling_on_sc=False),
    )
    def gather_k(table_hbm, idx_hbm, out_hbm, idx_v, rows_v, sem):
        wid = lax.axis_index("s") * NC + lax.axis_index("c")
        base = wid * b_per_w
        pltpu.sync_copy(idx_hbm.at[pl.ds(base, b_per_w)], idx_v)
        pltpu.async_copy(table_hbm.at[idx_v], rows_v, sem).wait()
        pltpu.sync_copy(rows_v, out_hbm.at[pl.ds(base, b_per_w)])

    return gather_k(table, idx)


_BM = 8
_NBUF = 12
_NTHREAD = 2
_NCHUNK = 11


def _tc_project(latent, W, b):
    B, D = latent.shape
    V = W.shape[0]
    Wt = W.T
    b2 = b.reshape(1, V)
    grid = B // _BM

    n_tiles = V // 128
    tail_lanes = V - n_tiles * 128
    bnd = [round(j * n_tiles / _NCHUNK) * 128 for j in range(_NCHUNK + 1)]
    chunks = [(bnd[j], bnd[j + 1] - bnd[j]) for j in range(_NCHUNK)]

    def body(lat_ref, wt_ref, b_ref, out_hbm, bufs, tails, sems):
        i = pl.program_id(0)
        slot = lax.rem(i, _NBUF)
        acc = lax.dot_general(
            lat_ref[pl.ds(i * _BM, _BM), :], wt_ref[...],
            (((1,), (0,)), ((), ())),
            preferred_element_type=jnp.float32,
        ) + b_ref[...]

        def slab_copies(step, sl):
            cps = [
                pltpu.make_async_copy(
                    bufs.at[sl, :, pl.ds(off, w)],
                    out_hbm.at[pl.ds(step * _BM, _BM), pl.ds(off, w)],
                    sems.at[sl],
                )
                for off, w in chunks
            ]
            if tail_lanes:
                cps.append(pltpu.make_async_copy(
                    tails.at[sl],
                    out_hbm.at[pl.ds(step * _BM, _BM),
                               pl.ds(n_tiles * 128, tail_lanes)],
                    sems.at[sl],
                ))
            return cps

        @pl.when(i >= _NBUF)
        def _():
            for cp in slab_copies(i - _NBUF, slot):
                cp.wait()

        bufs[slot] = acc
        if tail_lanes:
            tails[slot] = acc[:, n_tiles * 128:]
        for c, cp in enumerate(slab_copies(i, slot)):
            cp.start(priority=c % _NTHREAD)

        @pl.when(i == grid - 1)
        def _():
            for s in range(max(0, grid - _NBUF), grid):
                for cp in slab_copies(s, s % _NBUF):
                    cp.wait()

    return pl.pallas_call(
        body,
        grid=(grid,),
        in_specs=[
            pl.BlockSpec((B, D), lambda i: (0, 0)),
            pl.BlockSpec((D, V), lambda i: (0, 0)),
            pl.BlockSpec((1, V), lambda i: (0, 0)),
        ],
        out_specs=pl.BlockSpec(memory_space=pl.ANY),
        out_shape=jax.ShapeDtypeStruct((B, V), jnp.float32),
        scratch_shapes=[
            pltpu.VMEM((_NBUF, _BM, V), jnp.float32),
            pltpu.VMEM((_NBUF, _BM, max(tail_lanes, 1)), jnp.float32),
            pltpu.SemaphoreType.DMA((_NBUF,)),
        ],
        compiler_params=pltpu.CompilerParams(
            vmem_limit_bytes=110 * 1024 * 1024,
        ),
    )(latent, Wt, b2)


def kernel(inputs, emb_table, W, b):
    idx = inputs.astype(jnp.int32)
    latent = _sc_gather(emb_table, idx)
    return _tc_project(latent, W, b)


# speedup vs baseline: 2.1618x; 2.1375x over previous
import functools

import jax
import jax.numpy as jnp
from jax import lax
from jax.experimental import pallas as pl
from jax.experimental.pallas import tpu as pltpu
from jax.experimental.pallas import tpu_sc as plsc


def _sc_gather(table, idx):
    V, D = table.shape
    B = idx.shape[0]
    info = plsc.get_sparse_core_info()
    NC, NS = info.num_cores, info.num_subcores
    NW = NC * NS
    b_per_w = B // NW
    mesh = plsc.VectorSubcoreMesh(core_axis_name="c", subcore_axis_name="s")

    @functools.partial(
        pl.kernel,
        mesh=mesh,
        out_type=jax.ShapeDtypeStruct((B, D), jnp.float32),
        scratch_types=[
            pltpu.VMEM((b_per_w,), jnp.int32),
            pltpu.VMEM((b_per_w, D), jnp.float32),
            pltpu.SemaphoreType.DMA,
        ],
        compiler_params=pltpu.CompilerParams(use_tc_tiling_on_sc=False),
    )
    def gather_k(table_hbm, idx_hbm, out_hbm, idx_v, rows_v, sem):
        wid = lax.axis_index("s") * NC + lax.axis_index("c")
        base = wid * b_per_w
        pltpu.sync_copy(idx_hbm.at[pl.ds(base, b_per_w)], idx_v)
        pltpu.async_copy(table_hbm.at[idx_v], rows_v, sem).wait()
        pltpu.sync_copy(rows_v, out_hbm.at[pl.ds(base, b_per_w)])

    return gather_k(table, idx)


_BNV = 2048


def _matmul_body(wt_ref, lat_ref, b_ref, out_ref):
    out_ref[...] = lax.dot_general(
        wt_ref[...], lat_ref[...],
        (((0,), (1,)), ((), ())),
        preferred_element_type=jnp.float32,
    ) + b_ref[...]


def _tc_project(latent, W, b):
    B, D = latent.shape
    V = W.shape[0]
    Wt = W.T
    b2 = b.reshape(V, 1)
    grid = pl.cdiv(V, _BNV)
    return pl.pallas_call(
        _matmul_body,
        grid=(grid,),
        in_specs=[
            pl.BlockSpec((D, _BNV), lambda i: (0, i)),
            pl.BlockSpec((B, D), lambda i: (0, 0)),
            pl.BlockSpec((_BNV, 1), lambda i: (i, 0)),
        ],
        out_specs=pl.BlockSpec((_BNV, B), lambda i: (i, 0)),
        out_shape=jax.ShapeDtypeStruct((V, B), jnp.float32),
        compiler_params=pltpu.CompilerParams(
            vmem_limit_bytes=110 * 1024 * 1024,
        ),
    )(Wt, latent, b2)


def kernel(inputs, emb_table, W, b):
    idx = inputs.astype(jnp.int32)
    latent = _sc_gather(emb_table, idx)
    return _tc_project(latent, W, b).T


# speedup vs baseline: 2.7412x; 1.2680x over previous
import functools

import jax
import jax.numpy as jnp
from jax import lax
from jax.experimental import pallas as pl
from jax.experimental.pallas import tpu as pltpu
from jax.experimental.pallas import tpu_sc as plsc


def _sc_gather(table, idx):
    V, D = table.shape
    B = idx.shape[0]
    info = plsc.get_sparse_core_info()
    NC, NS = info.num_cores, info.num_subcores
    NW = NC * NS
    b_per_w = B // NW
    mesh = plsc.VectorSubcoreMesh(core_axis_name="c", subcore_axis_name="s")

    @functools.partial(
        pl.kernel,
        mesh=mesh,
        out_type=jax.ShapeDtypeStruct((B, D), jnp.float32),
        scratch_types=[
            pltpu.VMEM((b_per_w,), jnp.int32),
            pltpu.VMEM((b_per_w, D), jnp.float32),
            pltpu.SemaphoreType.DMA,
        ],
        compiler_params=pltpu.CompilerParams(use_tc_tiling_on_sc=False),
    )
    def gather_k(table_hbm, idx_hbm, out_hbm, idx_v, rows_v, sem):
        wid = lax.axis_index("s") * NC + lax.axis_index("c")
        base = wid * b_per_w
        pltpu.sync_copy(idx_hbm.at[pl.ds(base, b_per_w)], idx_v)
        pltpu.async_copy(table_hbm.at[idx_v], rows_v, sem).wait()
        pltpu.sync_copy(rows_v, out_hbm.at[pl.ds(base, b_per_w)])

    return gather_k(table, idx)


_BNV = 2048


def _matmul_body(wb_ref, lat_ref, out_ref):
    out_ref[...] = lax.dot_general(
        wb_ref[...], lat_ref[...],
        (((0,), (1,)), ((), ())),
        preferred_element_type=jnp.float32,
    )


def _tc_project(latent, W, b):
    B, D = latent.shape
    V = W.shape[0]
    wb = jnp.concatenate([W.T, b[None, :]], axis=0)
    lat_ext = jnp.concatenate(
        [latent, jnp.ones((B, 1), jnp.float32)], axis=1)
    grid = pl.cdiv(V, _BNV)
    return pl.pallas_call(
        _matmul_body,
        grid=(grid,),
        in_specs=[
            pl.BlockSpec((D + 1, _BNV), lambda i: (0, i)),
            pl.BlockSpec((B, D + 1), lambda i: (0, 0)),
        ],
        out_specs=pl.BlockSpec((_BNV, B), lambda i: (i, 0)),
        out_shape=jax.ShapeDtypeStruct((V, B), jnp.float32),
        compiler_params=pltpu.CompilerParams(
            vmem_limit_bytes=110 * 1024 * 1024,
        ),
    )(wb, lat_ext)


def kernel(inputs, emb_table, W, b):
    idx = inputs.astype(jnp.int32)
    latent = _sc_gather(emb_table, idx)
    return _tc_project(latent, W, b).T
